# baseline (speedup 1.0000x reference)
import jax
import jax.numpy as jnp
from jax.experimental import pallas as pl

N = 10000
K = 32
D = 128
BN = 400
BH = 200


def _fused_kernel(x_ref, na_ref, nb_ref, wlt_ref, wrt_ref, b_ref, o_ref):
    nsum_a = jnp.sum(na_ref[...], axis=1)
    nsum_b = jnp.sum(nb_ref[...], axis=1)
    nsum = jnp.concatenate([nsum_a, nsum_b], axis=0)
    acc = jnp.dot(x_ref[...], wlt_ref[...], preferred_element_type=jnp.float32)
    acc += jnp.dot(nsum * (1.0 / K), wrt_ref[...], preferred_element_type=jnp.float32)
    o_ref[...] = acc + b_ref[...]


@jax.jit
def kernel(x, neigh_x, W_l, b_l, W_r, b_r):
    wlt = W_l.T
    wrt = W_r.T
    b = (b_l + b_r).reshape(1, D)
    grid = (N // BN,)
    return pl.pallas_call(
        _fused_kernel,
        grid=grid,
        in_specs=[
            pl.BlockSpec((BN, D), lambda i: (i, 0)),
            pl.BlockSpec((BH, K, D), lambda i: (2 * i, 0, 0)),
            pl.BlockSpec((BH, K, D), lambda i: (2 * i + 1, 0, 0)),
            pl.BlockSpec((D, D), lambda i: (0, 0)),
            pl.BlockSpec((D, D), lambda i: (0, 0)),
            pl.BlockSpec((1, D), lambda i: (0, 0)),
        ],
        out_specs=pl.BlockSpec((BN, D), lambda i: (i, 0)),
        out_shape=jax.ShapeDtypeStruct((N, D), jnp.float32),
    )(x, neigh_x, neigh_x, wlt, wrt, b)


# speedup vs baseline: 1.0118x; 1.0118x over previous
import jax
import jax.numpy as jnp
from jax.experimental import pallas as pl

N = 10000
K = 32
D = 128
BN = 400


def _fused_kernel(x_ref, n_ref, wlt_ref, wrt_ref, b_ref, o_ref):
    nsum = jnp.sum(n_ref[...], axis=1)
    acc = jnp.dot(x_ref[...], wlt_ref[...], preferred_element_type=jnp.float32)
    acc += jnp.dot(nsum * (1.0 / K), wrt_ref[...], preferred_element_type=jnp.float32)
    o_ref[...] = acc + b_ref[...]


@jax.jit
def kernel(x, neigh_x, W_l, b_l, W_r, b_r):
    wlt = W_l.T
    wrt = W_r.T
    b = (b_l + b_r).reshape(1, D)
    grid = (N // BN,)
    return pl.pallas_call(
        _fused_kernel,
        grid=grid,
        in_specs=[
            pl.BlockSpec((BN, D), lambda i: (i, 0)),
            pl.BlockSpec((BN, K, D), lambda i: (i, 0, 0)),
            pl.BlockSpec((D, D), lambda i: (0, 0)),
            pl.BlockSpec((D, D), lambda i: (0, 0)),
            pl.BlockSpec((1, D), lambda i: (0, 0)),
        ],
        out_specs=pl.BlockSpec((BN, D), lambda i: (i, 0)),
        out_shape=jax.ShapeDtypeStruct((N, D), jnp.float32),
    )(x, neigh_x, wlt, wrt, b)
